# baseline (speedup 1.0000x reference)
import functools

import jax
import jax.numpy as jnp
from jax import lax
from jax.experimental import pallas as pl
from jax.experimental.pallas import tpu as pltpu
from jax.experimental.pallas import tpu_sc as plsc

DIM = 64
BATCH = 16384
SPLIT = 499968

_INFO = plsc.get_sparse_core_info()
_NC = _INFO.num_cores
_NS = _INFO.num_subcores
_NW = _NC * _NS
_PER_W = BATCH // _NW
_CHUNK = 256
_NCH = _PER_W // _CHUNK
_GCHUNK = 128
_NG = _PER_W // _GCHUNK
_EPS = 1e-12


def _gather_a_body(hidx_hbm, ridx_hbm, tidx_hbm, ent_hbm, rel_hbm,
                   hrows_hbm, rrows_hbm, trows_hbm,
                   idxh_v, idxr_v, idxt_v, bufh_v, bufr_v, buft_v,
                   semh, semr, semt):
    wid = lax.axis_index("s") * _NC + lax.axis_index("c")
    base = wid * _PER_W

    pltpu.sync_copy(hidx_hbm.at[pl.ds(base, _PER_W)], idxh_v)
    pltpu.sync_copy(ridx_hbm.at[pl.ds(base, _PER_W)], idxr_v)
    pltpu.sync_copy(tidx_hbm.at[pl.ds(base, _PER_W)], idxt_v)

    def make_fire(idx_v, tab_hbm, buf_v, sem, off):
        def fire(g, carry):
            v = idx_v[pl.ds(off + g * 16, 16)]
            for i in range(16):
                pltpu.async_copy(tab_hbm.at[pl.ds(v[i], 1)],
                                 buf_v.at[pl.ds(g * 16 + i, 1)], sem)
            return carry
        return fire

    for j in range(_NCH):
        off = j * _CHUNK
        lax.fori_loop(0, _CHUNK // 16, make_fire(idxh_v, ent_hbm, bufh_v, semh, off), 0)
        lax.fori_loop(0, _CHUNK // 16, make_fire(idxt_v, ent_hbm, buft_v, semt, off), 0)
        lax.fori_loop(0, _CHUNK // 16, make_fire(idxr_v, rel_hbm, bufr_v, semr, off), 0)
        dst = pl.ds(base + off, _CHUNK)
        pltpu.make_async_copy(ent_hbm.at[pl.ds(0, _CHUNK)], bufh_v, semh).wait()
        pltpu.sync_copy(bufh_v, hrows_hbm.at[dst])
        pltpu.make_async_copy(ent_hbm.at[pl.ds(0, _CHUNK)], buft_v, semt).wait()
        pltpu.sync_copy(buft_v, trows_hbm.at[dst])
        pltpu.make_async_copy(ent_hbm.at[pl.ds(0, _CHUNK)], bufr_v, semr).wait()
        pltpu.sync_copy(bufr_v, rrows_hbm.at[dst])


def _sc_gather_a(hidx, ridx, tidx, ent_lo, rel_emb):
    mesh = plsc.VectorSubcoreMesh(core_axis_name="c", subcore_axis_name="s")
    rows_t = jax.ShapeDtypeStruct((BATCH, DIM), jnp.float32)
    k = functools.partial(
        pl.kernel,
        mesh=mesh,
        out_type=[rows_t, rows_t, rows_t],
        scratch_types=[
            pltpu.VMEM((_PER_W,), jnp.int32),
            pltpu.VMEM((_PER_W,), jnp.int32),
            pltpu.VMEM((_PER_W,), jnp.int32),
            pltpu.VMEM((_CHUNK, DIM), jnp.float32),
            pltpu.VMEM((_CHUNK, DIM), jnp.float32),
            pltpu.VMEM((_CHUNK, DIM), jnp.float32),
            pltpu.SemaphoreType.DMA,
            pltpu.SemaphoreType.DMA,
            pltpu.SemaphoreType.DMA,
        ],
    )(_gather_a_body)
    return k(hidx, ridx, tidx, ent_lo, rel_emb)


def _gather_b_body(hidx_hbm, tidx_hbm, ent_hbm,
                   hrows_hbm, trows_hbm,
                   idxh_v, idxt_v, bufh_v, buft_v, semh, semt):
    wid = lax.axis_index("s") * _NC + lax.axis_index("c")

    pltpu.sync_copy(hidx_hbm.at[pl.ds(wid * _NG, _NG)], idxh_v)
    pltpu.sync_copy(tidx_hbm.at[pl.ds(wid * _NG, _NG)], idxt_v)

    for j in range(_NG):
        ch = pltpu.async_copy(ent_hbm.at[idxh_v.at[j]], bufh_v, semh)
        ct = pltpu.async_copy(ent_hbm.at[idxt_v.at[j]], buft_v, semt)
        dst = pl.ds(wid * _PER_W + j * _GCHUNK, _GCHUNK)
        ch.wait()
        pltpu.sync_copy(bufh_v, hrows_hbm.at[dst])
        ct.wait()
        pltpu.sync_copy(buft_v, trows_hbm.at[dst])


def _sc_gather_b(hidx2d, tidx2d, ent_hi):
    mesh = plsc.VectorSubcoreMesh(core_axis_name="c", subcore_axis_name="s")
    rows_t = jax.ShapeDtypeStruct((BATCH, DIM), jnp.float32)
    k = functools.partial(
        pl.kernel,
        mesh=mesh,
        compiler_params=pltpu.CompilerParams(use_tc_tiling_on_sc=False),
        out_type=[rows_t, rows_t],
        scratch_types=[
            pltpu.VMEM((_NG, _GCHUNK), jnp.int32),
            pltpu.VMEM((_NG, _GCHUNK), jnp.int32),
            pltpu.VMEM((_GCHUNK, DIM), jnp.float32),
            pltpu.VMEM((_GCHUNK, DIM), jnp.float32),
            pltpu.SemaphoreType.DMA,
            pltpu.SemaphoreType.DMA,
        ],
    )(_gather_b_body)
    return k(hidx2d, tidx2d, ent_hi)


def _score_body(ha_ref, ra_ref, ta_ref, hb_ref, tb_ref, mh_ref, mt_ref,
                o_ref):
    h = jnp.where(mh_ref[...] > 0, ha_ref[...], hb_ref[...])
    t = jnp.where(mt_ref[...] > 0, ta_ref[...], tb_ref[...])
    r = ra_ref[...]
    sh = jnp.maximum(jnp.sum(jnp.abs(h), axis=1, keepdims=True), _EPS)
    st = jnp.maximum(jnp.sum(jnp.abs(t), axis=1, keepdims=True), _EPS)
    d = jnp.abs(h / sh + r - t / st)
    o_ref[...] = -jnp.sum(d, axis=1)


def _tc_score(ha, ra, ta, hb, tb, mh, mt):
    blk = 1024
    grid = BATCH // blk
    spec = pl.BlockSpec((blk, DIM), lambda i: (i, 0))
    mspec = pl.BlockSpec((blk, 1), lambda i: (i, 0))
    return pl.pallas_call(
        _score_body,
        grid=(grid,),
        in_specs=[spec, spec, spec, spec, spec, mspec, mspec],
        out_specs=pl.BlockSpec((blk,), lambda i: (i,)),
        out_shape=jax.ShapeDtypeStruct((BATCH,), jnp.float32),
    )(ha, ra, ta, hb, tb, mh, mt)


def kernel(head_idxs, rel_idxs, tail_idxs, ent_emb, rel_emb):
    hidx = head_idxs.astype(jnp.int32)
    ridx = rel_idxs.astype(jnp.int32)
    tidx = tail_idxs.astype(jnp.int32)
    ent_lo = ent_emb[:SPLIT]
    ent_hi = ent_emb[SPLIT:]

    mh = hidx < SPLIT
    mt = tidx < SPLIT
    ha_idx = jnp.where(mh, hidx, 0)
    ta_idx = jnp.where(mt, tidx, 0)
    hb_idx = jnp.where(mh, 0, hidx - SPLIT).reshape(BATCH // _GCHUNK, _GCHUNK)
    tb_idx = jnp.where(mt, 0, tidx - SPLIT).reshape(BATCH // _GCHUNK, _GCHUNK)

    ha, ra, ta = _sc_gather_a(ha_idx, ridx, ta_idx, ent_lo, rel_emb)
    hb, tb = _sc_gather_b(hb_idx, tb_idx, ent_hi)
    return _tc_score(ha, ra, ta, hb, tb,
                     mh.astype(jnp.int32).reshape(BATCH, 1),
                     mt.astype(jnp.int32).reshape(BATCH, 1))


# speedup vs baseline: 4.2560x; 4.2560x over previous
import functools

import jax
import jax.numpy as jnp
from jax import lax
from jax.experimental import pallas as pl
from jax.experimental.pallas import tpu as pltpu
from jax.experimental.pallas import tpu_sc as plsc

DIM = 64
BATCH = 16384

_INFO = plsc.get_sparse_core_info()
_NC = _INFO.num_cores
_NS = _INFO.num_subcores
_NW = _NC * _NS
_PER_W = BATCH // _NW
_CHUNK = 256
_NCH = _PER_W // _CHUNK
_EPS = 1e-12


def _gather_body(hidx_hbm, ridx_hbm, tidx_hbm, ent_hbm, rel_hbm,
                 hrows_hbm, rrows_hbm, trows_hbm,
                 idxh_v, idxr_v, idxt_v, bufh_v, bufr_v, buft_v,
                 semh, semr, semt):
    wid = lax.axis_index("s") * _NC + lax.axis_index("c")
    base = wid * _PER_W

    pltpu.sync_copy(hidx_hbm.at[pl.ds(base, _PER_W)], idxh_v)
    pltpu.sync_copy(ridx_hbm.at[pl.ds(base, _PER_W)], idxr_v)
    pltpu.sync_copy(tidx_hbm.at[pl.ds(base, _PER_W)], idxt_v)

    def make_fire(idx_v, tab_hbm, buf_v, sem, off):
        def fire(g, carry):
            v = idx_v[pl.ds(off + g * 16, 16)]
            for i in range(16):
                pltpu.async_copy(tab_hbm.at[pl.ds(v[i], 1)],
                                 buf_v.at[pl.ds(g * 16 + i, 1)], sem)
            return carry
        return fire

    for j in range(_NCH):
        off = j * _CHUNK
        lax.fori_loop(0, _CHUNK // 16, make_fire(idxh_v, ent_hbm, bufh_v, semh, off), 0)
        lax.fori_loop(0, _CHUNK // 16, make_fire(idxt_v, ent_hbm, buft_v, semt, off), 0)
        lax.fori_loop(0, _CHUNK // 16, make_fire(idxr_v, rel_hbm, bufr_v, semr, off), 0)
        dst = pl.ds(base + off, _CHUNK)
        pltpu.make_async_copy(ent_hbm.at[pl.ds(0, _CHUNK)], bufh_v, semh).wait()
        pltpu.sync_copy(bufh_v, hrows_hbm.at[dst])
        pltpu.make_async_copy(ent_hbm.at[pl.ds(0, _CHUNK)], buft_v, semt).wait()
        pltpu.sync_copy(buft_v, trows_hbm.at[dst])
        pltpu.make_async_copy(ent_hbm.at[pl.ds(0, _CHUNK)], bufr_v, semr).wait()
        pltpu.sync_copy(bufr_v, rrows_hbm.at[dst])


def _sc_gather(hidx, ridx, tidx, ent_emb, rel_emb):
    mesh = plsc.VectorSubcoreMesh(core_axis_name="c", subcore_axis_name="s")
    rows_t = jax.ShapeDtypeStruct((BATCH, DIM), jnp.float32)
    k = functools.partial(
        pl.kernel,
        mesh=mesh,
        out_type=[rows_t, rows_t, rows_t],
        scratch_types=[
            pltpu.VMEM((_PER_W,), jnp.int32),
            pltpu.VMEM((_PER_W,), jnp.int32),
            pltpu.VMEM((_PER_W,), jnp.int32),
            pltpu.VMEM((_CHUNK, DIM), jnp.float32),
            pltpu.VMEM((_CHUNK, DIM), jnp.float32),
            pltpu.VMEM((_CHUNK, DIM), jnp.float32),
            pltpu.SemaphoreType.DMA,
            pltpu.SemaphoreType.DMA,
            pltpu.SemaphoreType.DMA,
        ],
    )(_gather_body)
    return k(hidx, ridx, tidx, ent_emb, rel_emb)


def _score_body(h_ref, r_ref, t_ref, o_ref):
    h = h_ref[...]
    r = r_ref[...]
    t = t_ref[...]
    sh = jnp.maximum(jnp.sum(jnp.abs(h), axis=1, keepdims=True), _EPS)
    st = jnp.maximum(jnp.sum(jnp.abs(t), axis=1, keepdims=True), _EPS)
    d = jnp.abs(h / sh + r - t / st)
    o_ref[...] = -jnp.sum(d, axis=1)


def _tc_score(hrows, rrows, trows):
    blk = 2048
    grid = BATCH // blk
    spec = pl.BlockSpec((blk, DIM), lambda i: (i, 0))
    return pl.pallas_call(
        _score_body,
        grid=(grid,),
        in_specs=[spec, spec, spec],
        out_specs=pl.BlockSpec((blk,), lambda i: (i,)),
        out_shape=jax.ShapeDtypeStruct((BATCH,), jnp.float32),
    )(hrows, rrows, trows)


def kernel(head_idxs, rel_idxs, tail_idxs, ent_emb, rel_emb):
    hidx = head_idxs.astype(jnp.int32)
    ridx = rel_idxs.astype(jnp.int32)
    tidx = tail_idxs.astype(jnp.int32)
    hrows, rrows, trows = _sc_gather(hidx, ridx, tidx, ent_emb, rel_emb)
    return _tc_score(hrows, rrows, trows)


# speedup vs baseline: 5.1090x; 1.2004x over previous
import functools

import jax
import jax.numpy as jnp
from jax import lax
from jax.experimental import pallas as pl
from jax.experimental.pallas import tpu as pltpu
from jax.experimental.pallas import tpu_sc as plsc

DIM = 64
BATCH = 16384

_INFO = plsc.get_sparse_core_info()
_NC = _INFO.num_cores
_NS = _INFO.num_subcores
_NW = _NC * _NS
_PER_W = BATCH // _NW
_CHUNK = 256
_NCH = _PER_W // _CHUNK
_EPS = 1e-12


def _gather_body(hidx_hbm, ridx_hbm, tidx_hbm, ent_hbm, rel_hbm,
                 hrows_hbm, rrows_hbm, trows_hbm,
                 idxh_v, idxr_v, idxt_v, bufh_v, bufr_v, buft_v,
                 semh, semr, semt):
    wid = lax.axis_index("s") * _NC + lax.axis_index("c")
    base = wid * _PER_W

    pltpu.sync_copy(hidx_hbm.at[pl.ds(base, _PER_W)], idxh_v)
    pltpu.sync_copy(ridx_hbm.at[pl.ds(base, _PER_W)], idxr_v)
    pltpu.sync_copy(tidx_hbm.at[pl.ds(base, _PER_W)], idxt_v)

    def make_fire(idx_v, tab_hbm, buf_v, sem, off):
        def fire(g, carry):
            v = idx_v[pl.ds(off + g * 16, 16)]
            for i in range(16):
                pltpu.async_copy(tab_hbm.at[pl.ds(v[i], 1)],
                                 buf_v.at[pl.ds(g * 16 + i, 1)], sem)
            return carry
        return fire

    for j in range(_NCH):
        off = j * _CHUNK
        lax.fori_loop(0, _CHUNK // 16, make_fire(idxh_v, ent_hbm, bufh_v, semh, off), 0)
        lax.fori_loop(0, _CHUNK // 16, make_fire(idxt_v, ent_hbm, buft_v, semt, off), 0)
        lax.fori_loop(0, _CHUNK // 16, make_fire(idxr_v, rel_hbm, bufr_v, semr, off), 0)
        dst = pl.ds(base + off, _CHUNK)
        pltpu.make_async_copy(ent_hbm.at[pl.ds(0, _CHUNK)], bufh_v, semh).wait()
        pltpu.sync_copy(bufh_v, hrows_hbm.at[dst])
        pltpu.make_async_copy(ent_hbm.at[pl.ds(0, _CHUNK)], buft_v, semt).wait()
        pltpu.sync_copy(buft_v, trows_hbm.at[dst])
        pltpu.make_async_copy(ent_hbm.at[pl.ds(0, _CHUNK)], bufr_v, semr).wait()
        pltpu.sync_copy(bufr_v, rrows_hbm.at[dst])


def _sc_gather(hidx, ridx, tidx, ent_emb, rel_emb):
    mesh = plsc.VectorSubcoreMesh(core_axis_name="c", subcore_axis_name="s")
    rows_t = jax.ShapeDtypeStruct((BATCH, DIM), jnp.float32)
    k = functools.partial(
        pl.kernel,
        mesh=mesh,
        out_type=[rows_t, rows_t, rows_t],
        scratch_types=[
            pltpu.VMEM((_PER_W,), jnp.int32),
            pltpu.VMEM((_PER_W,), jnp.int32),
            pltpu.VMEM((_PER_W,), jnp.int32),
            pltpu.VMEM((_CHUNK, DIM), jnp.float32),
            pltpu.VMEM((_CHUNK, DIM), jnp.float32),
            pltpu.VMEM((_CHUNK, DIM), jnp.float32),
            pltpu.SemaphoreType.DMA,
            pltpu.SemaphoreType.DMA,
            pltpu.SemaphoreType.DMA,
        ],
    )(_gather_body)
    return k(hidx, ridx, tidx, ent_emb, rel_emb)


_DK = 8192
_DG = -(-1000000 // _DK)


def _detile_body(xt_ref, o_ref):
    x = xt_ref[...]
    eye = jnp.eye(DIM, dtype=jnp.float32)
    o_ref[...] = jax.lax.dot_general(
        x, eye, (((0,), (0,)), ((), ())),
        preferred_element_type=jnp.float32)


def _tc_detile(entT):
    return pl.pallas_call(
        _detile_body,
        grid=(_DG,),
        in_specs=[pl.BlockSpec((DIM, _DK), lambda i: (0, i))],
        out_specs=pl.BlockSpec((_DK, DIM), lambda i: (i, 0)),
        out_shape=jax.ShapeDtypeStruct((entT.shape[1], DIM), jnp.float32),
    )(entT)


def _score_body(h_ref, r_ref, t_ref, o_ref):
    h = h_ref[...]
    r = r_ref[...]
    t = t_ref[...]
    sh = jnp.maximum(jnp.sum(jnp.abs(h), axis=1, keepdims=True), _EPS)
    st = jnp.maximum(jnp.sum(jnp.abs(t), axis=1, keepdims=True), _EPS)
    d = jnp.abs(h / sh + r - t / st)
    o_ref[...] = -jnp.sum(d, axis=1)


def _tc_score(hrows, rrows, trows):
    blk = 2048
    grid = BATCH // blk
    spec = pl.BlockSpec((blk, DIM), lambda i: (i, 0))
    return pl.pallas_call(
        _score_body,
        grid=(grid,),
        in_specs=[spec, spec, spec],
        out_specs=pl.BlockSpec((blk,), lambda i: (i,)),
        out_shape=jax.ShapeDtypeStruct((BATCH,), jnp.float32),
    )(hrows, rrows, trows)


def kernel(head_idxs, rel_idxs, tail_idxs, ent_emb, rel_emb):
    hidx = head_idxs.astype(jnp.int32)
    ridx = rel_idxs.astype(jnp.int32)
    tidx = tail_idxs.astype(jnp.int32)
    ent_rm = _tc_detile(ent_emb.T)
    hrows, rrows, trows = _sc_gather(hidx, ridx, tidx, ent_rm, rel_emb)
    return _tc_score(hrows, rrows, trows)


# speedup vs baseline: 5.5571x; 1.0877x over previous
import functools

import jax
import jax.numpy as jnp
from jax import lax
from jax.experimental import pallas as pl
from jax.experimental.pallas import tpu as pltpu
from jax.experimental.pallas import tpu_sc as plsc

DIM = 64
BATCH = 16384
NENT = 1000000

_KH = 8192
_NBLK = -(-NENT // (2 * _KH))
SPLITP = (_NBLK - 1) * _KH
_NROWS = _NBLK * _KH

_INFO = plsc.get_sparse_core_info()
_NC = _INFO.num_cores
_NS = _INFO.num_subcores
_NW = _NC * _NS
_PER_W = BATCH // _NW
_CHUNK = 256
_NCH = _PER_W // _CHUNK
_EPS = 1e-12


def _detile_body(x1_ref, x2_ref, o_ref):
    eye = jnp.eye(DIM, dtype=jnp.float32)
    dims = (((0,), (0,)), ((), ()))
    y1 = jax.lax.dot_general(x1_ref[...], eye, dims,
                             preferred_element_type=jnp.float32)
    y2 = jax.lax.dot_general(x2_ref[...], eye, dims,
                             preferred_element_type=jnp.float32)
    o_ref[...] = jnp.concatenate([y1, y2], axis=1)


def _tc_detile(entT):
    return pl.pallas_call(
        _detile_body,
        grid=(_NBLK,),
        in_specs=[pl.BlockSpec((DIM, _KH), lambda i: (0, i)),
                  pl.BlockSpec((DIM, _KH), lambda i: (0, i + _NBLK - 1))],
        out_specs=pl.BlockSpec((_KH, 2 * DIM), lambda i: (i, 0)),
        out_shape=jax.ShapeDtypeStruct((_NROWS, 2 * DIM), jnp.float32),
    )(entT, entT)


def _gather_body(hidx_hbm, ridx_hbm, tidx_hbm, ent_hbm, rel_hbm,
                 hrows_hbm, rrows_hbm, trows_hbm,
                 idxh_v, idxr_v, idxt_v, bufh_v, bufr_v, buft_v,
                 semh, semr, semt):
    wid = lax.axis_index("s") * _NC + lax.axis_index("c")
    base = wid * _PER_W

    pltpu.sync_copy(hidx_hbm.at[pl.ds(base, _PER_W)], idxh_v)
    pltpu.sync_copy(ridx_hbm.at[pl.ds(base, _PER_W)], idxr_v)
    pltpu.sync_copy(tidx_hbm.at[pl.ds(base, _PER_W)], idxt_v)

    def make_fire(idx_v, tab_hbm, buf_v, sem, off):
        def fire(g, carry):
            v = idx_v[pl.ds(off + g * 16, 16)]
            for i in range(16):
                pltpu.async_copy(tab_hbm.at[pl.ds(v[i], 1)],
                                 buf_v.at[pl.ds(g * 16 + i, 1)], sem)
            return carry
        return fire

    for j in range(_NCH):
        off = j * _CHUNK
        lax.fori_loop(0, _CHUNK // 16, make_fire(idxh_v, ent_hbm, bufh_v, semh, off), 0)
        lax.fori_loop(0, _CHUNK // 16, make_fire(idxt_v, ent_hbm, buft_v, semt, off), 0)
        lax.fori_loop(0, _CHUNK // 16, make_fire(idxr_v, rel_hbm, bufr_v, semr, off), 0)
        dst = pl.ds(base + off, _CHUNK)
        pltpu.make_async_copy(ent_hbm.at[pl.ds(0, _CHUNK)], bufh_v, semh).wait()
        pltpu.sync_copy(bufh_v, hrows_hbm.at[dst])
        pltpu.make_async_copy(ent_hbm.at[pl.ds(0, _CHUNK)], buft_v, semt).wait()
        pltpu.sync_copy(buft_v, trows_hbm.at[dst])
        pltpu.make_async_copy(rel_hbm.at[pl.ds(0, _CHUNK)], bufr_v, semr).wait()
        pltpu.sync_copy(bufr_v, rrows_hbm.at[dst])


def _sc_gather(hidx, ridx, tidx, ent2, rel_emb):
    mesh = plsc.VectorSubcoreMesh(core_axis_name="c", subcore_axis_name="s")
    line_t = jax.ShapeDtypeStruct((BATCH, 2 * DIM), jnp.float32)
    rows_t = jax.ShapeDtypeStruct((BATCH, DIM), jnp.float32)
    k = functools.partial(
        pl.kernel,
        mesh=mesh,
        out_type=[line_t, rows_t, line_t],
        scratch_types=[
            pltpu.VMEM((_PER_W,), jnp.int32),
            pltpu.VMEM((_PER_W,), jnp.int32),
            pltpu.VMEM((_PER_W,), jnp.int32),
            pltpu.VMEM((_CHUNK, 2 * DIM), jnp.float32),
            pltpu.VMEM((_CHUNK, DIM), jnp.float32),
            pltpu.VMEM((_CHUNK, 2 * DIM), jnp.float32),
            pltpu.SemaphoreType.DMA,
            pltpu.SemaphoreType.DMA,
            pltpu.SemaphoreType.DMA,
        ],
    )(_gather_body)
    return k(hidx, ridx, tidx, ent2, rel_emb)


def _score_body(h2_ref, r_ref, t2_ref, mh_ref, mt_ref, o_ref):
    h2 = h2_ref[...]
    t2 = t2_ref[...]
    r = r_ref[...]
    h = jnp.where(mh_ref[...] > 0, h2[:, DIM:], h2[:, :DIM])
    t = jnp.where(mt_ref[...] > 0, t2[:, DIM:], t2[:, :DIM])
    sh = jnp.maximum(jnp.sum(jnp.abs(h), axis=1, keepdims=True), _EPS)
    st = jnp.maximum(jnp.sum(jnp.abs(t), axis=1, keepdims=True), _EPS)
    d = jnp.abs(h / sh + r - t / st)
    o_ref[...] = -jnp.sum(d, axis=1)


def _tc_score(h2, r, t2, mh, mt):
    blk = 2048
    grid = BATCH // blk
    lspec = pl.BlockSpec((blk, 2 * DIM), lambda i: (i, 0))
    rspec = pl.BlockSpec((blk, DIM), lambda i: (i, 0))
    mspec = pl.BlockSpec((blk, 1), lambda i: (i, 0))
    return pl.pallas_call(
        _score_body,
        grid=(grid,),
        in_specs=[lspec, rspec, lspec, mspec, mspec],
        out_specs=pl.BlockSpec((blk,), lambda i: (i,)),
        out_shape=jax.ShapeDtypeStruct((BATCH,), jnp.float32),
    )(h2, r, t2, mh, mt)


def kernel(head_idxs, rel_idxs, tail_idxs, ent_emb, rel_emb):
    hidx = head_idxs.astype(jnp.int32)
    ridx = rel_idxs.astype(jnp.int32)
    tidx = tail_idxs.astype(jnp.int32)
    ent2 = _tc_detile(ent_emb.T)
    mh = hidx >= SPLITP
    mt = tidx >= SPLITP
    h2idx = jnp.where(mh, hidx - SPLITP, hidx)
    t2idx = jnp.where(mt, tidx - SPLITP, tidx)
    h2, rrows, t2rows = _sc_gather(h2idx, ridx, t2idx, ent2, rel_emb)
    return _tc_score(h2, rrows, t2rows,
                     mh.astype(jnp.int32).reshape(BATCH, 1),
                     mt.astype(jnp.int32).reshape(BATCH, 1))
